# Initial kernel scaffold; baseline (speedup 1.0000x reference)
import jax
import jax.numpy as jnp
from jax.experimental import pallas as pl


def kernel(features, edge_index, W1, b1, W2, b2):
    raise NotImplementedError("write your pallas kernel here")


# speedup vs baseline: 8.3970x; 8.3970x over previous
import functools

import jax
import jax.numpy as jnp
from jax import lax
from jax.experimental import pallas as pl
from jax.experimental.pallas import tpu as pltpu
from jax.experimental.pallas import tpu_sc as plsc

N_NODES = 10000
N_EDGES = 320000
F_IN = 128
HID = 16
N_CLASSES = 40

NC, NS = 2, 16
NW = NC * NS
N_PAD = 10240
ROWS_T = N_PAD // NS
CH = 128
NCH = 80
E_W = NCH * CH
E_PAD = NW * E_W
C_PAD = 48

_mesh = plsc.VectorSubcoreMesh(core_axis_name="c", subcore_axis_name="s",
                               num_cores=NC, num_subcores=NS)
_sc_params = pltpu.CompilerParams(use_tc_tiling_on_sc=False)


@functools.partial(
    pl.kernel,
    out_type=(
        jax.ShapeDtypeStruct((NC, N_PAD), jnp.float32),
        jax.ShapeDtypeStruct((NC, N_PAD), jnp.float32),
    ),
    mesh=_mesh,
    scratch_types=[
        pltpu.VMEM((NCH, CH), jnp.int32),
        pltpu.VMEM((NCH, CH), jnp.int32),
        pltpu.VMEM((CH,), jnp.float32),
        pltpu.VMEM_SHARED((N_PAD,), jnp.float32),
        pltpu.VMEM_SHARED((N_PAD,), jnp.float32),
    ],
    compiler_params=_sc_params,
)
def _sc_degrees(src_hbm, dst_hbm, ones_hbm, zeros_hbm,
                degout_hbm, degin_hbm,
                src_v, dst_v, ones_v, acc_o, acc_i):
    c = lax.axis_index("c")
    s = lax.axis_index("s")
    w = c * NS + s
    sl = pl.ds(s * ROWS_T, ROWS_T)
    pltpu.sync_copy(zeros_hbm.at[sl], acc_o.at[sl])
    pltpu.sync_copy(zeros_hbm.at[sl], acc_i.at[sl])
    pltpu.sync_copy(ones_hbm, ones_v)
    pltpu.sync_copy(src_hbm.at[w], src_v)
    pltpu.sync_copy(dst_hbm.at[w], dst_v)
    plsc.subcore_barrier()

    def body(i, carry):
        pltpu.sync_copy(ones_v, acc_o.at[src_v.at[i]], add=True)
        pltpu.sync_copy(ones_v, acc_i.at[dst_v.at[i]], add=True)
        return carry

    lax.fori_loop(0, NCH, body, 0)
    plsc.subcore_barrier()
    pltpu.sync_copy(acc_o.at[sl], degout_hbm.at[c, sl])
    pltpu.sync_copy(acc_i.at[sl], degin_hbm.at[c, sl])


def _make_sc_aggregate(d):
    @functools.partial(
        pl.kernel,
        out_type=jax.ShapeDtypeStruct((NC, N_PAD, d), jnp.float32),
        mesh=_mesh,
        scratch_types=[
            pltpu.VMEM((NCH, CH), jnp.int32),
            pltpu.VMEM((NCH, CH), jnp.int32),
            pltpu.VMEM((CH, d), jnp.float32),
            pltpu.VMEM_SHARED((N_PAD, d), jnp.float32),
            pltpu.SemaphoreType.DMA,
        ],
        compiler_params=_sc_params,
    )
    def sc_agg(rows_hbm, src_hbm, dst_hbm, zeros_hbm, out_hbm,
               src_v, dst_v, rows_v, acc, sem):
        c = lax.axis_index("c")
        s = lax.axis_index("s")
        w = c * NS + s
        sl = pl.ds(s * ROWS_T, ROWS_T)
        pltpu.sync_copy(zeros_hbm.at[sl], acc.at[sl])
        pltpu.sync_copy(src_hbm.at[w], src_v)
        pltpu.sync_copy(dst_hbm.at[w], dst_v)
        plsc.subcore_barrier()

        def body(i, carry):
            pltpu.async_copy(rows_hbm.at[src_v.at[i]], rows_v, sem).wait()
            pltpu.sync_copy(rows_v, acc.at[dst_v.at[i]], add=True)
            return carry

        lax.fori_loop(0, NCH, body, 0)
        plsc.subcore_barrier()
        pltpu.sync_copy(acc.at[sl], out_hbm.at[c, sl])

    return sc_agg


_sc_agg16 = _make_sc_aggregate(HID)
_sc_agg48 = _make_sc_aggregate(C_PAD)


_B = 1024


def _tc_mm1_body(x_ref, w_ref, do_ref, o_ref):
    deg = do_ref[0] + do_ref[1]
    norm = lax.rsqrt(jnp.maximum(deg, 1.0))
    o_ref[...] = jnp.dot(x_ref[...] * norm, w_ref[...],
                         preferred_element_type=jnp.float32)


def _tc_mm1(x_pad, w1, degout_p):
    return pl.pallas_call(
        _tc_mm1_body,
        grid=(N_PAD // _B,),
        in_specs=[
            pl.BlockSpec((_B, F_IN), lambda i: (i, 0)),
            pl.BlockSpec((F_IN, HID), lambda i: (0, 0)),
            pl.BlockSpec((NC, _B, 1), lambda i: (0, i, 0)),
        ],
        out_specs=pl.BlockSpec((_B, HID), lambda i: (i, 0)),
        out_shape=jax.ShapeDtypeStruct((N_PAD, HID), jnp.float32),
    )(x_pad, w1, degout_p)


def _tc_mm2_body(a_ref, di_ref, do_ref, b1_ref, w2_ref, o_ref):
    agg = a_ref[0] + a_ref[1]
    ndst = lax.rsqrt(jnp.maximum(di_ref[0] + di_ref[1], 1.0))
    h = jax.nn.relu(agg * ndst + b1_ref[...])
    nsrc = lax.rsqrt(jnp.maximum(do_ref[0] + do_ref[1], 1.0))
    o_ref[...] = jnp.dot(h * nsrc, w2_ref[...],
                         preferred_element_type=jnp.float32)


def _tc_mm2(agg1_p, degin_p, degout_p, b1, w2p):
    return pl.pallas_call(
        _tc_mm2_body,
        grid=(N_PAD // _B,),
        in_specs=[
            pl.BlockSpec((NC, _B, HID), lambda i: (0, i, 0)),
            pl.BlockSpec((NC, _B, 1), lambda i: (0, i, 0)),
            pl.BlockSpec((NC, _B, 1), lambda i: (0, i, 0)),
            pl.BlockSpec((1, HID), lambda i: (0, 0)),
            pl.BlockSpec((HID, C_PAD), lambda i: (0, 0)),
        ],
        out_specs=pl.BlockSpec((_B, C_PAD), lambda i: (i, 0)),
        out_shape=jax.ShapeDtypeStruct((N_PAD, C_PAD), jnp.float32),
    )(agg1_p, degin_p, degout_p, b1, w2p)


def _tc_finish_body(a_ref, di_ref, b2_ref, o_ref):
    agg = a_ref[0] + a_ref[1]
    ndst = lax.rsqrt(jnp.maximum(di_ref[0] + di_ref[1], 1.0))
    o_ref[...] = agg * ndst + b2_ref[...]


def _tc_finish(agg2_p, degin_p, b2p):
    return pl.pallas_call(
        _tc_finish_body,
        grid=(N_PAD // _B,),
        in_specs=[
            pl.BlockSpec((NC, _B, C_PAD), lambda i: (0, i, 0)),
            pl.BlockSpec((NC, _B, 1), lambda i: (0, i, 0)),
            pl.BlockSpec((1, C_PAD), lambda i: (0, 0)),
        ],
        out_specs=pl.BlockSpec((_B, C_PAD), lambda i: (i, 0)),
        out_shape=jax.ShapeDtypeStruct((N_PAD, C_PAD), jnp.float32),
    )(agg2_p, degin_p, b2p)


def kernel(features, edge_index, W1, b1, W2, b2):
    pad_e = E_PAD - N_EDGES
    src = jnp.pad(edge_index[0], (0, pad_e), constant_values=N_PAD - 1)
    dst = jnp.pad(edge_index[1], (0, pad_e), constant_values=N_PAD - 1)
    src3 = src.reshape(NW, NCH, CH)
    dst3 = dst.reshape(NW, NCH, CH)
    x_pad = jnp.pad(features, ((0, N_PAD - N_NODES), (0, 0)))
    w2p = jnp.pad(W2, ((0, 0), (0, C_PAD - N_CLASSES)))
    b2p = jnp.pad(b2, (0, C_PAD - N_CLASSES)).reshape(1, C_PAD)
    b1r = b1.reshape(1, HID)

    ones_ch = jnp.ones((CH,), jnp.float32)
    zeros_1d = jnp.zeros((N_PAD,), jnp.float32)
    zeros_16 = jnp.zeros((N_PAD, HID), jnp.float32)
    zeros_48 = jnp.zeros((N_PAD, C_PAD), jnp.float32)

    degout_p, degin_p = _sc_degrees(src3, dst3, ones_ch, zeros_1d)
    degout_p = degout_p.reshape(NC, N_PAD, 1)
    degin_p = degin_p.reshape(NC, N_PAD, 1)

    xw = _tc_mm1(x_pad, W1, degout_p)
    agg1_p = _sc_agg16(xw, src3, dst3, zeros_16)
    hw = _tc_mm2(agg1_p, degin_p, degout_p, b1r, w2p)
    agg2_p = _sc_agg48(hw, src3, dst3, zeros_48)
    out = _tc_finish(agg2_p, degin_p, b2p)
    return out[:N_NODES, :N_CLASSES]


# speedup vs baseline: 10.3248x; 1.2296x over previous
import functools

import jax
import jax.numpy as jnp
from jax import lax
from jax.experimental import pallas as pl
from jax.experimental.pallas import tpu as pltpu
from jax.experimental.pallas import tpu_sc as plsc

N_NODES = 10000
N_EDGES = 320000
F_IN = 128
HID = 16
N_CLASSES = 40

NC, NS = 2, 16
NW = NC * NS
N_PAD = 10240
ROWS_T = N_PAD // NS
CH = 128
NCH = 80
E_W = NCH * CH
E_PAD = NW * E_W
C_PAD = 48

_mesh = plsc.VectorSubcoreMesh(core_axis_name="c", subcore_axis_name="s",
                               num_cores=NC, num_subcores=NS)
_sc_params = pltpu.CompilerParams(use_tc_tiling_on_sc=False)


@functools.partial(
    pl.kernel,
    out_type=(
        jax.ShapeDtypeStruct((NC, N_PAD), jnp.float32),
        jax.ShapeDtypeStruct((NC, N_PAD), jnp.float32),
    ),
    mesh=_mesh,
    scratch_types=[
        pltpu.VMEM((NCH, CH), jnp.int32),
        pltpu.VMEM((NCH, CH), jnp.int32),
        pltpu.VMEM((CH,), jnp.float32),
        pltpu.VMEM_SHARED((N_PAD,), jnp.float32),
        pltpu.VMEM_SHARED((N_PAD,), jnp.float32),
        pltpu.SemaphoreType.DMA,
    ],
    compiler_params=_sc_params,
)
def _sc_degrees(src_hbm, dst_hbm, ones_hbm, zeros_hbm,
                degout_hbm, degin_hbm,
                src_v, dst_v, ones_v, acc_o, acc_i, ssem):
    c = lax.axis_index("c")
    s = lax.axis_index("s")
    w = c * NS + s
    sl = pl.ds(s * ROWS_T, ROWS_T)
    pltpu.sync_copy(zeros_hbm.at[sl], acc_o.at[sl])
    pltpu.sync_copy(zeros_hbm.at[sl], acc_i.at[sl])
    pltpu.sync_copy(ones_hbm, ones_v)
    pltpu.sync_copy(src_hbm.at[w], src_v)
    pltpu.sync_copy(dst_hbm.at[w], dst_v)
    plsc.subcore_barrier()

    depth = 4

    def wait_s():
        pltpu.make_async_copy(ones_v, acc_o.at[src_v.at[0]], ssem).wait()

    def wait_s2():
        wait_s()
        wait_s()

    def body(i, carry):
        pl.when(i >= depth)(wait_s2)
        pltpu.async_copy(ones_v, acc_o.at[src_v.at[i]], ssem, add=True)
        pltpu.async_copy(ones_v, acc_i.at[dst_v.at[i]], ssem, add=True)
        return carry

    lax.fori_loop(0, NCH, body, 0)
    for _ in range(2 * depth):
        wait_s()
    plsc.subcore_barrier()
    pltpu.sync_copy(acc_o.at[sl], degout_hbm.at[c, sl])
    pltpu.sync_copy(acc_i.at[sl], degin_hbm.at[c, sl])


def _make_sc_aggregate(d):
    nbuf, d_g, d_s = 6, 3, 3

    @functools.partial(
        pl.kernel,
        out_type=jax.ShapeDtypeStruct((NC, N_PAD, d), jnp.float32),
        mesh=_mesh,
        scratch_types=[
            pltpu.VMEM((NCH, CH), jnp.int32),
            pltpu.VMEM((NCH, CH), jnp.int32),
            pltpu.VMEM((nbuf, CH, d), jnp.float32),
            pltpu.VMEM_SHARED((N_PAD, d), jnp.float32),
            pltpu.SemaphoreType.DMA,
            pltpu.SemaphoreType.DMA,
        ],
        compiler_params=_sc_params,
    )
    def sc_agg(rows_hbm, src_hbm, dst_hbm, zeros_hbm, out_hbm,
               src_v, dst_v, rows_v, acc, gsem, ssem):
        c = lax.axis_index("c")
        s = lax.axis_index("s")
        w = c * NS + s
        sl = pl.ds(s * ROWS_T, ROWS_T)
        pltpu.sync_copy(zeros_hbm.at[sl], acc.at[sl])
        pltpu.sync_copy(src_hbm.at[w], src_v)
        pltpu.sync_copy(dst_hbm.at[w], dst_v)
        plsc.subcore_barrier()

        def start_g(i):
            pltpu.async_copy(rows_hbm.at[src_v.at[i]],
                             rows_v.at[lax.rem(i, nbuf)], gsem)

        def wait_g():
            pltpu.make_async_copy(rows_hbm.at[src_v.at[0]],
                                  rows_v.at[0], gsem).wait()

        def start_s(i):
            pltpu.async_copy(rows_v.at[lax.rem(i, nbuf)],
                             acc.at[dst_v.at[i]], ssem, add=True)

        def wait_s():
            pltpu.make_async_copy(rows_v.at[0],
                                  acc.at[dst_v.at[0]], ssem).wait()

        for i in range(d_g):
            start_g(i)

        def body(i, carry):
            wait_g()
            pl.when(i >= d_s)(wait_s)
            pl.when(i + d_g < NCH)(lambda: start_g(i + d_g))
            start_s(i)
            return carry

        lax.fori_loop(0, NCH, body, 0)
        for _ in range(d_s):
            wait_s()
        plsc.subcore_barrier()
        pltpu.sync_copy(acc.at[sl], out_hbm.at[c, sl])

    return sc_agg


_sc_agg16 = _make_sc_aggregate(HID)
_sc_agg48 = _make_sc_aggregate(C_PAD)


_B = 1024


def _tc_mm1_body(x_ref, w_ref, do_ref, o_ref):
    deg = do_ref[0] + do_ref[1]
    norm = lax.rsqrt(jnp.maximum(deg, 1.0))
    o_ref[...] = jnp.dot(x_ref[...] * norm, w_ref[...],
                         preferred_element_type=jnp.float32)


def _tc_mm1(x_pad, w1, degout_p):
    return pl.pallas_call(
        _tc_mm1_body,
        grid=(N_PAD // _B,),
        in_specs=[
            pl.BlockSpec((_B, F_IN), lambda i: (i, 0)),
            pl.BlockSpec((F_IN, HID), lambda i: (0, 0)),
            pl.BlockSpec((NC, _B, 1), lambda i: (0, i, 0)),
        ],
        out_specs=pl.BlockSpec((_B, HID), lambda i: (i, 0)),
        out_shape=jax.ShapeDtypeStruct((N_PAD, HID), jnp.float32),
    )(x_pad, w1, degout_p)


def _tc_mm2_body(a_ref, di_ref, do_ref, b1_ref, w2_ref, o_ref):
    agg = a_ref[0] + a_ref[1]
    ndst = lax.rsqrt(jnp.maximum(di_ref[0] + di_ref[1], 1.0))
    h = jax.nn.relu(agg * ndst + b1_ref[...])
    nsrc = lax.rsqrt(jnp.maximum(do_ref[0] + do_ref[1], 1.0))
    o_ref[...] = jnp.dot(h * nsrc, w2_ref[...],
                         preferred_element_type=jnp.float32)


def _tc_mm2(agg1_p, degin_p, degout_p, b1, w2p):
    return pl.pallas_call(
        _tc_mm2_body,
        grid=(N_PAD // _B,),
        in_specs=[
            pl.BlockSpec((NC, _B, HID), lambda i: (0, i, 0)),
            pl.BlockSpec((NC, _B, 1), lambda i: (0, i, 0)),
            pl.BlockSpec((NC, _B, 1), lambda i: (0, i, 0)),
            pl.BlockSpec((1, HID), lambda i: (0, 0)),
            pl.BlockSpec((HID, C_PAD), lambda i: (0, 0)),
        ],
        out_specs=pl.BlockSpec((_B, C_PAD), lambda i: (i, 0)),
        out_shape=jax.ShapeDtypeStruct((N_PAD, C_PAD), jnp.float32),
    )(agg1_p, degin_p, degout_p, b1, w2p)


def _tc_finish_body(a_ref, di_ref, b2_ref, o_ref):
    agg = a_ref[0] + a_ref[1]
    ndst = lax.rsqrt(jnp.maximum(di_ref[0] + di_ref[1], 1.0))
    o_ref[...] = agg * ndst + b2_ref[...]


def _tc_finish(agg2_p, degin_p, b2p):
    return pl.pallas_call(
        _tc_finish_body,
        grid=(N_PAD // _B,),
        in_specs=[
            pl.BlockSpec((NC, _B, C_PAD), lambda i: (0, i, 0)),
            pl.BlockSpec((NC, _B, 1), lambda i: (0, i, 0)),
            pl.BlockSpec((1, C_PAD), lambda i: (0, 0)),
        ],
        out_specs=pl.BlockSpec((_B, C_PAD), lambda i: (i, 0)),
        out_shape=jax.ShapeDtypeStruct((N_PAD, C_PAD), jnp.float32),
    )(agg2_p, degin_p, b2p)


def kernel(features, edge_index, W1, b1, W2, b2):
    pad_e = E_PAD - N_EDGES
    src = jnp.pad(edge_index[0], (0, pad_e), constant_values=N_PAD - 1)
    dst = jnp.pad(edge_index[1], (0, pad_e), constant_values=N_PAD - 1)
    src3 = src.reshape(NW, NCH, CH)
    dst3 = dst.reshape(NW, NCH, CH)
    x_pad = jnp.pad(features, ((0, N_PAD - N_NODES), (0, 0)))
    w2p = jnp.pad(W2, ((0, 0), (0, C_PAD - N_CLASSES)))
    b2p = jnp.pad(b2, (0, C_PAD - N_CLASSES)).reshape(1, C_PAD)
    b1r = b1.reshape(1, HID)

    ones_ch = jnp.ones((CH,), jnp.float32)
    zeros_1d = jnp.zeros((N_PAD,), jnp.float32)
    zeros_16 = jnp.zeros((N_PAD, HID), jnp.float32)
    zeros_48 = jnp.zeros((N_PAD, C_PAD), jnp.float32)

    degout_p, degin_p = _sc_degrees(src3, dst3, ones_ch, zeros_1d)
    degout_p = degout_p.reshape(NC, N_PAD, 1)
    degin_p = degin_p.reshape(NC, N_PAD, 1)

    xw = _tc_mm1(x_pad, W1, degout_p)
    agg1_p = _sc_agg16(xw, src3, dst3, zeros_16)
    hw = _tc_mm2(agg1_p, degin_p, degout_p, b1r, w2p)
    agg2_p = _sc_agg48(hw, src3, dst3, zeros_48)
    out = _tc_finish(agg2_p, degin_p, b2p)
    return out[:N_NODES, :N_CLASSES]


# speedup vs baseline: 18.3078x; 1.7732x over previous
import functools

import jax
import jax.numpy as jnp
from jax import lax
from jax.experimental import pallas as pl
from jax.experimental.pallas import tpu as pltpu
from jax.experimental.pallas import tpu_sc as plsc

N_NODES = 10000
N_EDGES = 320000
F_IN = 128
HID = 16
N_CLASSES = 40

NC, NS = 2, 16
NW = NC * NS
N_PAD = 10240
ROWS_T = N_PAD // NS
CH = 128
NCH = 80
E_W = NCH * CH
E_PAD = NW * E_W
C_PAD = 48

_mesh = plsc.VectorSubcoreMesh(core_axis_name="c", subcore_axis_name="s",
                               num_cores=NC, num_subcores=NS)
_sc_params = pltpu.CompilerParams(use_tc_tiling_on_sc=False)


@functools.partial(
    pl.kernel,
    out_type=(
        jax.ShapeDtypeStruct((NC, N_PAD), jnp.float32),
        jax.ShapeDtypeStruct((NC, N_PAD), jnp.float32),
    ),
    mesh=_mesh,
    scratch_types=[
        pltpu.VMEM((NCH, CH), jnp.int32),
        pltpu.VMEM((NCH, CH), jnp.int32),
        pltpu.VMEM((CH,), jnp.float32),
        pltpu.VMEM_SHARED((N_PAD,), jnp.float32),
        pltpu.VMEM_SHARED((N_PAD,), jnp.float32),
        pltpu.SemaphoreType.DMA,
    ],
    compiler_params=_sc_params,
)
def _sc_degrees(src_hbm, dst_hbm, ones_hbm, zeros_hbm,
                degout_hbm, degin_hbm,
                src_v, dst_v, ones_v, acc_o, acc_i, ssem):
    c = lax.axis_index("c")
    s = lax.axis_index("s")
    w = c * NS + s
    sl = pl.ds(s * ROWS_T, ROWS_T)
    pltpu.sync_copy(zeros_hbm.at[sl], acc_o.at[sl])
    pltpu.sync_copy(zeros_hbm.at[sl], acc_i.at[sl])
    pltpu.sync_copy(ones_hbm, ones_v)
    pltpu.sync_copy(src_hbm.at[w], src_v)
    pltpu.sync_copy(dst_hbm.at[w], dst_v)
    plsc.subcore_barrier()

    depth = 8

    def wait_s():
        pltpu.make_async_copy(ones_v, acc_o.at[src_v.at[0]], ssem).wait()

    def wait_s2():
        wait_s()
        wait_s()

    def body(i, carry):
        pl.when(i >= depth)(wait_s2)
        pltpu.async_copy(ones_v, acc_o.at[src_v.at[i]], ssem, add=True)
        pltpu.async_copy(ones_v, acc_i.at[dst_v.at[i]], ssem, add=True)
        return carry

    lax.fori_loop(0, NCH, body, 0)
    for _ in range(2 * depth):
        wait_s()
    plsc.subcore_barrier()
    pltpu.sync_copy(acc_o.at[sl], degout_hbm.at[c, sl])
    pltpu.sync_copy(acc_i.at[sl], degin_hbm.at[c, sl])


def _make_sc_aggregate(d):
    nbuf, d_g, d_s = 8, 4, 4

    @functools.partial(
        pl.kernel,
        out_type=jax.ShapeDtypeStruct((NC, N_PAD, d), jnp.float32),
        mesh=_mesh,
        scratch_types=[
            pltpu.VMEM((NCH, CH), jnp.int32),
            pltpu.VMEM((NCH, CH), jnp.int32),
            pltpu.VMEM((nbuf, CH, d), jnp.float32),
            pltpu.VMEM_SHARED((N_PAD, d), jnp.float32),
            pltpu.SemaphoreType.DMA,
            pltpu.SemaphoreType.DMA,
        ],
        compiler_params=_sc_params,
    )
    def sc_agg(rows_hbm, src_hbm, dst_hbm, zeros_hbm, out_hbm,
               src_v, dst_v, rows_v, acc, gsem, ssem):
        c = lax.axis_index("c")
        s = lax.axis_index("s")
        w = c * NS + s
        sl = pl.ds(s * ROWS_T, ROWS_T)
        pltpu.sync_copy(zeros_hbm.at[sl], acc.at[sl])
        pltpu.sync_copy(src_hbm.at[w], src_v)
        pltpu.sync_copy(dst_hbm.at[w], dst_v)
        plsc.subcore_barrier()

        def start_g(i):
            pltpu.async_copy(rows_hbm.at[src_v.at[i]],
                             rows_v.at[lax.rem(i, nbuf)], gsem)

        def wait_g():
            pltpu.make_async_copy(rows_hbm.at[src_v.at[0]],
                                  rows_v.at[0], gsem).wait()

        def start_s(i):
            pltpu.async_copy(rows_v.at[lax.rem(i, nbuf)],
                             acc.at[dst_v.at[i]], ssem, add=True)

        def wait_s():
            pltpu.make_async_copy(rows_v.at[0],
                                  acc.at[dst_v.at[0]], ssem).wait()

        for i in range(d_g):
            start_g(i)

        def body(i, carry):
            wait_g()
            pl.when(i >= d_s)(wait_s)
            pl.when(i + d_g < NCH)(lambda: start_g(i + d_g))
            start_s(i)
            return carry

        lax.fori_loop(0, NCH, body, 0)
        for _ in range(d_s):
            wait_s()
        plsc.subcore_barrier()
        pltpu.sync_copy(acc.at[sl], out_hbm.at[c, sl])

    return sc_agg


_sc_agg16 = _make_sc_aggregate(HID)
_sc_agg48 = _make_sc_aggregate(C_PAD)


_B = 1024


def _tc_mm1_body(x_ref, w_ref, do_ref, o_ref):
    deg = do_ref[0] + do_ref[1]
    norm = lax.rsqrt(jnp.maximum(deg, 1.0))
    o_ref[...] = jnp.dot(x_ref[...] * norm, w_ref[...],
                         preferred_element_type=jnp.float32)


def _tc_mm1(x_pad, w1, degout_p):
    return pl.pallas_call(
        _tc_mm1_body,
        grid=(N_PAD // _B,),
        in_specs=[
            pl.BlockSpec((_B, F_IN), lambda i: (i, 0)),
            pl.BlockSpec((F_IN, HID), lambda i: (0, 0)),
            pl.BlockSpec((NC, _B, 1), lambda i: (0, i, 0)),
        ],
        out_specs=pl.BlockSpec((_B, HID), lambda i: (i, 0)),
        out_shape=jax.ShapeDtypeStruct((N_PAD, HID), jnp.float32),
    )(x_pad, w1, degout_p)


def _tc_mm2_body(a_ref, di_ref, do_ref, b1_ref, w2_ref, o_ref):
    agg = a_ref[0] + a_ref[1]
    ndst = lax.rsqrt(jnp.maximum(di_ref[0] + di_ref[1], 1.0))
    h = jax.nn.relu(agg * ndst + b1_ref[...])
    nsrc = lax.rsqrt(jnp.maximum(do_ref[0] + do_ref[1], 1.0))
    o_ref[...] = jnp.dot(h * nsrc, w2_ref[...],
                         preferred_element_type=jnp.float32)


def _tc_mm2(agg1_p, degin_p, degout_p, b1, w2p):
    return pl.pallas_call(
        _tc_mm2_body,
        grid=(N_PAD // _B,),
        in_specs=[
            pl.BlockSpec((NC, _B, HID), lambda i: (0, i, 0)),
            pl.BlockSpec((NC, _B, 1), lambda i: (0, i, 0)),
            pl.BlockSpec((NC, _B, 1), lambda i: (0, i, 0)),
            pl.BlockSpec((1, HID), lambda i: (0, 0)),
            pl.BlockSpec((HID, C_PAD), lambda i: (0, 0)),
        ],
        out_specs=pl.BlockSpec((_B, C_PAD), lambda i: (i, 0)),
        out_shape=jax.ShapeDtypeStruct((N_PAD, C_PAD), jnp.float32),
    )(agg1_p, degin_p, degout_p, b1, w2p)


def _tc_finish_body(a_ref, di_ref, b2_ref, o_ref):
    agg = a_ref[0] + a_ref[1]
    ndst = lax.rsqrt(jnp.maximum(di_ref[0] + di_ref[1], 1.0))
    o_ref[...] = agg * ndst + b2_ref[...]


def _tc_finish(agg2_p, degin_p, b2p):
    return pl.pallas_call(
        _tc_finish_body,
        grid=(N_PAD // _B,),
        in_specs=[
            pl.BlockSpec((NC, _B, C_PAD), lambda i: (0, i, 0)),
            pl.BlockSpec((NC, _B, 1), lambda i: (0, i, 0)),
            pl.BlockSpec((1, C_PAD), lambda i: (0, 0)),
        ],
        out_specs=pl.BlockSpec((_B, C_PAD), lambda i: (i, 0)),
        out_shape=jax.ShapeDtypeStruct((N_PAD, C_PAD), jnp.float32),
    )(agg2_p, degin_p, b2p)


def kernel(features, edge_index, W1, b1, W2, b2):
    pad_e = E_PAD - N_EDGES
    pad_idx = (jnp.arange(pad_e, dtype=jnp.int32) % (N_PAD - N_NODES)) + N_NODES
    src = jnp.concatenate([edge_index[0], pad_idx])
    dst = jnp.concatenate([edge_index[1], pad_idx])
    src3 = src.reshape(NW, NCH, CH)
    dst3 = dst.reshape(NW, NCH, CH)
    x_pad = jnp.pad(features, ((0, N_PAD - N_NODES), (0, 0)))
    w2p = jnp.pad(W2, ((0, 0), (0, C_PAD - N_CLASSES)))
    b2p = jnp.pad(b2, (0, C_PAD - N_CLASSES)).reshape(1, C_PAD)
    b1r = b1.reshape(1, HID)

    ones_ch = jnp.ones((CH,), jnp.float32)
    zeros_1d = jnp.zeros((N_PAD,), jnp.float32)
    zeros_16 = jnp.zeros((N_PAD, HID), jnp.float32)
    zeros_48 = jnp.zeros((N_PAD, C_PAD), jnp.float32)

    degout_p, degin_p = _sc_degrees(src3, dst3, ones_ch, zeros_1d)
    degout_p = degout_p.reshape(NC, N_PAD, 1)
    degin_p = degin_p.reshape(NC, N_PAD, 1)

    xw = _tc_mm1(x_pad, W1, degout_p)
    agg1_p = _sc_agg16(xw, src3, dst3, zeros_16)
    hw = _tc_mm2(agg1_p, degin_p, degout_p, b1r, w2p)
    agg2_p = _sc_agg48(hw, src3, dst3, zeros_48)
    out = _tc_finish(agg2_p, degin_p, b2p)
    return out[:N_NODES, :N_CLASSES]


# speedup vs baseline: 22.2545x; 1.2156x over previous
import functools

import jax
import jax.numpy as jnp
from jax import lax
from jax.experimental import pallas as pl
from jax.experimental.pallas import tpu as pltpu
from jax.experimental.pallas import tpu_sc as plsc

N_NODES = 10000
N_EDGES = 320000
F_IN = 128
HID = 16
N_CLASSES = 40

NC, NS = 2, 16
NW = NC * NS
N_PAD = 10240
ROWS_T = N_PAD // NS
ROWS_W = N_PAD // NW
CH = 128
NCH = 80
E_PAD = NW * NCH * CH
C_PAD = 48

_mesh = plsc.VectorSubcoreMesh(core_axis_name="c", subcore_axis_name="s",
                               num_cores=NC, num_subcores=NS)
_sc_params = pltpu.CompilerParams(use_tc_tiling_on_sc=False)


def _rsqrt16(d):
    x = jnp.maximum(d, 1.0)
    i = lax.bitcast_convert_type(x, jnp.int32)
    i = jnp.int32(0x5F3759DF) - lax.shift_right_arithmetic(i, 1)
    y = lax.bitcast_convert_type(i, jnp.float32)
    for _ in range(4):
        y = y * (1.5 - 0.5 * x * y * y)
    return y


@functools.partial(
    pl.kernel,
    out_type=(
        jax.ShapeDtypeStruct((NC, N_PAD), jnp.float32),
        jax.ShapeDtypeStruct((NC, N_PAD), jnp.float32),
    ),
    mesh=_mesh,
    scratch_types=[
        pltpu.VMEM((NCH, CH), jnp.int32),
        pltpu.VMEM((NCH, CH), jnp.int32),
        pltpu.VMEM((CH,), jnp.float32),
        pltpu.VMEM_SHARED((N_PAD,), jnp.float32),
        pltpu.VMEM_SHARED((N_PAD,), jnp.float32),
        pltpu.SemaphoreType.DMA,
    ],
    compiler_params=_sc_params,
)
def _sc_degrees(src_hbm, dst_hbm, ones_hbm, zeros_hbm,
                degout_hbm, degin_hbm,
                src_v, dst_v, ones_v, acc_o, acc_i, ssem):
    c = lax.axis_index("c")
    s = lax.axis_index("s")
    w = c * NS + s
    sl = pl.ds(s * ROWS_T, ROWS_T)
    pltpu.sync_copy(zeros_hbm.at[sl], acc_o.at[sl])
    pltpu.sync_copy(zeros_hbm.at[sl], acc_i.at[sl])
    pltpu.sync_copy(ones_hbm, ones_v)
    pltpu.sync_copy(src_hbm.at[w], src_v)
    pltpu.sync_copy(dst_hbm.at[w], dst_v)
    plsc.subcore_barrier()

    depth = 8

    def wait_s():
        pltpu.make_async_copy(ones_v, acc_o.at[src_v.at[0]], ssem).wait()

    def wait_s2():
        wait_s()
        wait_s()

    def body(i, carry):
        pl.when(i >= depth)(wait_s2)
        pltpu.async_copy(ones_v, acc_o.at[src_v.at[i]], ssem, add=True)
        pltpu.async_copy(ones_v, acc_i.at[dst_v.at[i]], ssem, add=True)
        return carry

    lax.fori_loop(0, NCH, body, 0)
    for _ in range(2 * depth):
        wait_s()
    plsc.subcore_barrier()
    pltpu.sync_copy(acc_o.at[sl], degout_hbm.at[c, sl])
    pltpu.sync_copy(acc_i.at[sl], degin_hbm.at[c, sl])


_NBUF, _DG, _DS = 8, 4, 4


def _agg_pipeline(gather_src_at, src_v, dst_v, rows_v, acc, gsem, ssem):
    def start_g(i):
        pltpu.async_copy(gather_src_at(src_v.at[i]),
                         rows_v.at[lax.rem(i, _NBUF)], gsem)

    def wait_g():
        pltpu.make_async_copy(gather_src_at(src_v.at[0]),
                              rows_v.at[0], gsem).wait()

    def start_s(i):
        pltpu.async_copy(rows_v.at[lax.rem(i, _NBUF)],
                         acc.at[dst_v.at[i]], ssem, add=True)

    def wait_s():
        pltpu.make_async_copy(rows_v.at[0], acc.at[dst_v.at[0]], ssem).wait()

    for i in range(_DG):
        start_g(i)

    def body(i, carry):
        wait_g()
        pl.when(i >= _DS)(wait_s)
        pl.when(i + _DG < NCH)(lambda: start_g(i + _DG))
        start_s(i)
        return carry

    lax.fori_loop(0, NCH, body, 0)
    for _ in range(_DS):
        wait_s()


@functools.partial(
    pl.kernel,
    out_type=jax.ShapeDtypeStruct((NC, N_PAD, HID), jnp.float32),
    mesh=_mesh,
    scratch_types=[
        pltpu.VMEM((NCH, CH), jnp.int32),
        pltpu.VMEM((NCH, CH), jnp.int32),
        pltpu.VMEM((_NBUF, CH, HID), jnp.float32),
        pltpu.VMEM((ROWS_T, HID), jnp.float32),
        pltpu.VMEM((ROWS_T,), jnp.float32),
        pltpu.VMEM((ROWS_T,), jnp.float32),
        pltpu.VMEM((ROWS_T,), jnp.float32),
        pltpu.VMEM_SHARED((N_PAD, HID), jnp.float32),
        pltpu.VMEM_SHARED((N_PAD, HID), jnp.float32),
        pltpu.SemaphoreType.DMA,
        pltpu.SemaphoreType.DMA,
    ],
    compiler_params=_sc_params,
)
def _sc_agg16(xwun_hbm, degs_hbm, src_hbm, dst_hbm, zeros_hbm, out_hbm,
              src_v, dst_v, rows_v, xrows_v, dg0_v, dg1_v, norm_v,
              table, acc, gsem, ssem):
    c = lax.axis_index("c")
    s = lax.axis_index("s")
    w = c * NS + s
    sl = pl.ds(s * ROWS_T, ROWS_T)

    pltpu.sync_copy(xwun_hbm.at[sl], xrows_v)
    pltpu.sync_copy(degs_hbm.at[0, sl], dg0_v)
    pltpu.sync_copy(degs_hbm.at[1, sl], dg1_v)
    pltpu.sync_copy(zeros_hbm.at[sl], acc.at[sl])
    pltpu.sync_copy(src_hbm.at[w], src_v)
    pltpu.sync_copy(dst_hbm.at[w], dst_v)

    def nbody(j, carry):
        d = dg0_v[pl.ds(j * 16, 16)] + dg1_v[pl.ds(j * 16, 16)]
        norm_v[pl.ds(j * 16, 16)] = _rsqrt16(d)
        return carry

    lax.fori_loop(0, ROWS_T // 16, nbody, 0)

    def sbody(j, carry):
        nv = norm_v[pl.ds(j * 16, 16)]
        for t in range(16):
            i = j * 16 + t
            xrows_v[i, :] = xrows_v[i, :] * nv[t]
        return carry

    lax.fori_loop(0, ROWS_T // 16, sbody, 0)
    pltpu.sync_copy(xrows_v, table.at[sl])
    plsc.subcore_barrier()

    _agg_pipeline(lambda idx: table.at[idx],
                  src_v, dst_v, rows_v, acc, gsem, ssem)
    plsc.subcore_barrier()
    pltpu.sync_copy(acc.at[sl], out_hbm.at[c, sl])


@functools.partial(
    pl.kernel,
    out_type=jax.ShapeDtypeStruct((NC, N_PAD, C_PAD), jnp.float32),
    mesh=_mesh,
    scratch_types=[
        pltpu.VMEM((NCH, CH), jnp.int32),
        pltpu.VMEM((NCH, CH), jnp.int32),
        pltpu.VMEM((_NBUF, CH, C_PAD), jnp.float32),
        pltpu.VMEM_SHARED((N_PAD, C_PAD), jnp.float32),
        pltpu.SemaphoreType.DMA,
        pltpu.SemaphoreType.DMA,
    ],
    compiler_params=_sc_params,
)
def _sc_agg48(rows_hbm, src_hbm, dst_hbm, zeros_hbm, out_hbm,
              src_v, dst_v, rows_v, acc, gsem, ssem):
    c = lax.axis_index("c")
    s = lax.axis_index("s")
    w = c * NS + s
    sl = pl.ds(s * ROWS_T, ROWS_T)
    pltpu.sync_copy(zeros_hbm.at[sl], acc.at[sl])
    pltpu.sync_copy(src_hbm.at[w], src_v)
    pltpu.sync_copy(dst_hbm.at[w], dst_v)
    plsc.subcore_barrier()

    _agg_pipeline(lambda idx: rows_hbm.at[idx],
                  src_v, dst_v, rows_v, acc, gsem, ssem)
    plsc.subcore_barrier()
    pltpu.sync_copy(acc.at[sl], out_hbm.at[c, sl])


@functools.partial(
    pl.kernel,
    out_type=jax.ShapeDtypeStruct((N_PAD, HID), jnp.float32),
    mesh=_mesh,
    scratch_types=[
        pltpu.VMEM((ROWS_W, HID), jnp.float32),
        pltpu.VMEM((ROWS_W, HID), jnp.float32),
        pltpu.VMEM((ROWS_W,), jnp.float32),
        pltpu.VMEM((ROWS_W,), jnp.float32),
        pltpu.VMEM((ROWS_W,), jnp.float32),
        pltpu.VMEM((ROWS_W,), jnp.float32),
        pltpu.VMEM((ROWS_W,), jnp.float32),
        pltpu.VMEM((ROWS_W,), jnp.float32),
        pltpu.VMEM((HID,), jnp.float32),
    ],
    compiler_params=_sc_params,
)
def _sc_fin16(aggp_hbm, degin_hbm, degout_hbm, b1_hbm, h_hbm,
              p0_v, p1_v, di0_v, di1_v, do0_v, do1_v, nd_v, ns_v, b_v):
    c = lax.axis_index("c")
    s = lax.axis_index("s")
    w = c * NS + s
    slw = pl.ds(w * ROWS_W, ROWS_W)
    pltpu.sync_copy(aggp_hbm.at[0, slw], p0_v)
    pltpu.sync_copy(aggp_hbm.at[1, slw], p1_v)
    pltpu.sync_copy(degin_hbm.at[0, slw], di0_v)
    pltpu.sync_copy(degin_hbm.at[1, slw], di1_v)
    pltpu.sync_copy(degout_hbm.at[0, slw], do0_v)
    pltpu.sync_copy(degout_hbm.at[1, slw], do1_v)
    pltpu.sync_copy(b1_hbm, b_v)

    def nbody(j, carry):
        ds = pl.ds(j * 16, 16)
        nd_v[ds] = _rsqrt16(di0_v[ds] + di1_v[ds])
        ns_v[ds] = _rsqrt16(do0_v[ds] + do1_v[ds])
        return carry

    lax.fori_loop(0, ROWS_W // 16, nbody, 0)

    def rbody(j, carry):
        ndv = nd_v[pl.ds(j * 16, 16)]
        nsv = ns_v[pl.ds(j * 16, 16)]
        for t in range(16):
            i = j * 16 + t
            row = (p0_v[i, :] + p1_v[i, :]) * ndv[t] + b_v[:]
            p0_v[i, :] = jnp.maximum(row, 0.0) * nsv[t]
        return carry

    lax.fori_loop(0, ROWS_W // 16, rbody, 0)
    pltpu.sync_copy(p0_v, h_hbm.at[slw])


@functools.partial(
    pl.kernel,
    out_type=jax.ShapeDtypeStruct((N_PAD, C_PAD), jnp.float32),
    mesh=_mesh,
    scratch_types=[
        pltpu.VMEM((ROWS_W, C_PAD), jnp.float32),
        pltpu.VMEM((ROWS_W, C_PAD), jnp.float32),
        pltpu.VMEM((ROWS_W,), jnp.float32),
        pltpu.VMEM((ROWS_W,), jnp.float32),
        pltpu.VMEM((ROWS_W,), jnp.float32),
        pltpu.VMEM((C_PAD,), jnp.float32),
    ],
    compiler_params=_sc_params,
)
def _sc_fin48(aggp_hbm, degin_hbm, b2_hbm, out_hbm,
              p0_v, p1_v, di0_v, di1_v, nd_v, b_v):
    c = lax.axis_index("c")
    s = lax.axis_index("s")
    w = c * NS + s
    slw = pl.ds(w * ROWS_W, ROWS_W)
    pltpu.sync_copy(aggp_hbm.at[0, slw], p0_v)
    pltpu.sync_copy(aggp_hbm.at[1, slw], p1_v)
    pltpu.sync_copy(degin_hbm.at[0, slw], di0_v)
    pltpu.sync_copy(degin_hbm.at[1, slw], di1_v)
    pltpu.sync_copy(b2_hbm, b_v)

    def nbody(j, carry):
        ds = pl.ds(j * 16, 16)
        nd_v[ds] = _rsqrt16(di0_v[ds] + di1_v[ds])
        return carry

    lax.fori_loop(0, ROWS_W // 16, nbody, 0)

    def rbody(j, carry):
        ndv = nd_v[pl.ds(j * 16, 16)]
        for t in range(16):
            i = j * 16 + t
            for k in range(C_PAD // 16):
                ds = pl.ds(k * 16, 16)
                p0_v[i, ds] = (p0_v[i, ds] + p1_v[i, ds]) * ndv[t] + b_v[ds]
        return carry

    lax.fori_loop(0, ROWS_W // 16, rbody, 0)
    pltpu.sync_copy(p0_v, out_hbm.at[slw])


_B = 1024


def _tc_mm_body(x_ref, w_ref, o_ref):
    o_ref[...] = jnp.dot(x_ref[...], w_ref[...],
                         preferred_element_type=jnp.float32)


def _tc_mm(x, w):
    m, k = x.shape
    n = w.shape[1]
    return pl.pallas_call(
        _tc_mm_body,
        grid=(m // _B,),
        in_specs=[
            pl.BlockSpec((_B, k), lambda i: (i, 0)),
            pl.BlockSpec((k, n), lambda i: (0, 0)),
        ],
        out_specs=pl.BlockSpec((_B, n), lambda i: (i, 0)),
        out_shape=jax.ShapeDtypeStruct((m, n), jnp.float32),
    )(x, w)


def kernel(features, edge_index, W1, b1, W2, b2):
    pad_e = E_PAD - N_EDGES
    pad_idx = (jnp.arange(pad_e, dtype=jnp.int32) % (N_PAD - N_NODES)) + N_NODES
    src3 = jnp.concatenate([edge_index[0], pad_idx]).reshape(NW, NCH, CH)
    dst3 = jnp.concatenate([edge_index[1], pad_idx]).reshape(NW, NCH, CH)
    x_pad = jnp.pad(features, ((0, N_PAD - N_NODES), (0, 0)))
    w2p = jnp.pad(W2, ((0, 0), (0, C_PAD - N_CLASSES)))
    b2p = jnp.pad(b2, (0, C_PAD - N_CLASSES))

    ones_ch = jnp.ones((CH,), jnp.float32)
    zeros_1d = jnp.zeros((N_PAD,), jnp.float32)
    zeros_16 = jnp.zeros((N_PAD, HID), jnp.float32)
    zeros_48 = jnp.zeros((N_PAD, C_PAD), jnp.float32)

    degout_p, degin_p = _sc_degrees(src3, dst3, ones_ch, zeros_1d)
    xw_un = _tc_mm(x_pad, W1)
    agg1_p = _sc_agg16(xw_un, degout_p, src3, dst3, zeros_16)
    h = _sc_fin16(agg1_p, degin_p, degout_p, b1)
    hw = _tc_mm(h, w2p)
    agg2_p = _sc_agg48(hw, src3, dst3, zeros_48)
    out = _sc_fin48(agg2_p, degin_p, b2p)
    return out[:N_NODES, :N_CLASSES]
